# Initial kernel scaffold; baseline (speedup 1.0000x reference)
import functools

import jax
import jax.numpy as jnp
from jax import lax
from jax.experimental import pallas as pl
from jax.experimental.pallas import tpu as pltpu
from jax.experimental.pallas import tpu_sc as plsc

NN = 10000
NP = 10240
EE = 320000
NTILES = 32
EPT = EE // NTILES
CH = 80
NCH = EPT // CH
RPT = NP // 16
DW = 8

_MESH = plsc.VectorSubcoreMesh(core_axis_name="c", subcore_axis_name="s")


def _make_edge_sum(D):
    @functools.partial(
        pl.kernel,
        out_type=jax.ShapeDtypeStruct((2, NP, D), jnp.float32),
        mesh=_MESH,
        scratch_types=[
            pltpu.VMEM((NCH, CH), jnp.int32),
            pltpu.VMEM((NCH, CH), jnp.int32),
            pltpu.VMEM((CH, D), jnp.float32),
            pltpu.VMEM_SHARED((NP, D), jnp.float32),
            pltpu.SemaphoreType.DMA,
        ],
    )
    def edge_sum(src_hbm, dst_hbm, y_hbm, zeros_hbm, out_hbm,
                 src_v, dst_v, rows_v, acc, sem):
        c = lax.axis_index("c")
        s = lax.axis_index("s")
        w = c * 16 + s
        z0 = s * RPT
        pltpu.sync_copy(zeros_hbm.at[pl.ds(z0, RPT)], acc.at[pl.ds(z0, RPT)])
        pltpu.sync_copy(src_hbm.at[w], src_v)
        pltpu.sync_copy(dst_hbm.at[w], dst_v)
        plsc.subcore_barrier()

        def body(i, carry):
            pltpu.async_copy(y_hbm.at[src_v.at[i]], rows_v, sem).wait()
            pltpu.sync_copy(rows_v, acc.at[dst_v.at[i]], add=True)
            return carry

        lax.fori_loop(0, NCH, body, 0)
        plsc.subcore_barrier()
        pltpu.sync_copy(acc.at[pl.ds(z0, RPT)], out_hbm.at[c, pl.ds(z0, RPT)])

    return edge_sum


@functools.partial(
    pl.kernel,
    out_type=jax.ShapeDtypeStruct((2, NP, DW), jnp.float32),
    mesh=_MESH,
    scratch_types=[
        pltpu.VMEM((NCH, CH), jnp.int32),
        pltpu.VMEM((CH, DW), jnp.float32),
        pltpu.VMEM_SHARED((NP, DW), jnp.float32),
    ],
)
def _degree(dst_hbm, zeros_hbm, ones_hbm, out_hbm, dst_v, ones_v, acc):
    c = lax.axis_index("c")
    s = lax.axis_index("s")
    w = c * 16 + s
    z0 = s * RPT
    pltpu.sync_copy(zeros_hbm.at[pl.ds(z0, RPT)], acc.at[pl.ds(z0, RPT)])
    pltpu.sync_copy(dst_hbm.at[w], dst_v)
    pltpu.sync_copy(ones_hbm, ones_v)
    plsc.subcore_barrier()

    def body(i, carry):
        pltpu.sync_copy(ones_v, acc.at[dst_v.at[i]], add=True)
        return carry

    lax.fori_loop(0, NCH, body, 0)
    plsc.subcore_barrier()
    pltpu.sync_copy(acc.at[pl.ds(z0, RPT)], out_hbm.at[c, pl.ds(z0, RPT)])


_edge_sum_128 = _make_edge_sum(128)
_edge_sum_64 = _make_edge_sum(64)


_BLK = 1000
_GRID = (NN // _BLK,)


def _row_spec(d):
    return pl.BlockSpec((_BLK, d), lambda i: (i, 0))


def _full_spec(r, c):
    return pl.BlockSpec((r, c), lambda i: (0, 0))


def _prescale_body(x_ref, w1_ref, c0_ref, c1_ref, y_ref, dinv_ref):
    deg = c0_ref[...] + c1_ref[...] + 1.0
    dinv = lax.rsqrt(deg)
    y_ref[...] = jnp.dot(x_ref[...], w1_ref[...],
                         preferred_element_type=jnp.float32) * dinv
    dinv_ref[...] = dinv


def _tc_prescale(x, w1, cnt0, cnt1):
    return pl.pallas_call(
        _prescale_body,
        grid=_GRID,
        in_specs=[_row_spec(128), _full_spec(128, 128), _row_spec(1), _row_spec(1)],
        out_specs=[_row_spec(128), _row_spec(1)],
        out_shape=[jax.ShapeDtypeStruct((NN, 128), jnp.float32),
                   jax.ShapeDtypeStruct((NN, 1), jnp.float32)],
    )(x, w1, cnt0, cnt1)


def _mid_body(a0_ref, a1_ref, y1_ref, dinv_ref, b1_ref, w2_ref, y2_ref):
    dinv = dinv_ref[...]
    h1 = dinv * (a0_ref[...] + a1_ref[...] + y1_ref[...]) + b1_ref[...]
    h1 = jnp.maximum(h1, 0.0)
    y2_ref[...] = jnp.dot(h1, w2_ref[...],
                          preferred_element_type=jnp.float32) * dinv


def _tc_mid(a0, a1, y1, dinv, b1, w2):
    return pl.pallas_call(
        _mid_body,
        grid=_GRID,
        in_specs=[_row_spec(128), _row_spec(128), _row_spec(128), _row_spec(1),
                  _full_spec(1, 128), _full_spec(128, 64)],
        out_specs=_row_spec(64),
        out_shape=jax.ShapeDtypeStruct((NN, 64), jnp.float32),
    )(a0, a1, y1, dinv, b1, w2)


def _final_body(a0_ref, a1_ref, y2_ref, dinv_ref, b2_ref, wc_ref, bc_ref,
                h2_ref, out_ref):
    h2 = dinv_ref[...] * (a0_ref[...] + a1_ref[...] + y2_ref[...]) + b2_ref[...]
    h2_ref[...] = h2
    out_ref[...] = jnp.dot(h2, wc_ref[...],
                           preferred_element_type=jnp.float32) + bc_ref[...]


def _tc_final(a0, a1, y2, dinv, b2, wc, bc):
    return pl.pallas_call(
        _final_body,
        grid=_GRID,
        in_specs=[_row_spec(64), _row_spec(64), _row_spec(64), _row_spec(1),
                  _full_spec(1, 64), _full_spec(64, 64), _full_spec(1, 64)],
        out_specs=[_row_spec(64), _row_spec(64)],
        out_shape=[jax.ShapeDtypeStruct((NN, 64), jnp.float32),
                   jax.ShapeDtypeStruct((NN, 64), jnp.float32)],
    )(a0, a1, y2, dinv, b2, wc, bc)


def kernel(x, edge_index, W1, b1, W2, b2, Wc, bc):
    src_r = edge_index[0].reshape(NTILES, NCH, CH)
    dst_r = edge_index[1].reshape(NTILES, NCH, CH)

    zeros8 = jnp.zeros((NP, DW), jnp.float32)
    ones8 = jnp.ones((CH, DW), jnp.float32)
    zeros128 = jnp.zeros((NP, 128), jnp.float32)
    zeros64 = jnp.zeros((NP, 64), jnp.float32)

    deg_o = _degree(dst_r, zeros8, ones8)
    cnt0 = deg_o[0, :NN, 0:1]
    cnt1 = deg_o[1, :NN, 0:1]

    y1, dinv = _tc_prescale(x, W1, cnt0, cnt1)

    acc1 = _edge_sum_128(src_r, dst_r, y1, zeros128)
    y2 = _tc_mid(acc1[0, :NN], acc1[1, :NN], y1, dinv, b1.reshape(1, 128), W2)

    acc2 = _edge_sum_64(src_r, dst_r, y2, zeros64)
    out, h2 = _tc_final(acc2[0, :NN], acc2[1, :NN], y2, dinv,
                        b2.reshape(1, 64), Wc, bc.reshape(1, 64))
    return (out, h2)


# speedup vs baseline: 19.7575x; 19.7575x over previous
import functools

import jax
import jax.numpy as jnp
from jax import lax
from jax.experimental import pallas as pl
from jax.experimental.pallas import tpu as pltpu
from jax.experimental.pallas import tpu_sc as plsc

NN = 10000
NP = 10240
EE = 320000
NTILES = 32
EPT = EE // NTILES
CH = 80
NCH = EPT // CH
RPT = NP // 16
DW = 8

_MESH = plsc.VectorSubcoreMesh(core_axis_name="c", subcore_axis_name="s")


def _make_edge_sum(D):
    @functools.partial(
        pl.kernel,
        out_type=jax.ShapeDtypeStruct((2, NP, D), jnp.float32),
        mesh=_MESH,
        compiler_params=pltpu.CompilerParams(use_tc_tiling_on_sc=False),
        scratch_types=[
            pltpu.VMEM((NCH, CH), jnp.int32),
            pltpu.VMEM((NCH, CH), jnp.int32),
            pltpu.VMEM((CH, D), jnp.float32),
            pltpu.VMEM_SHARED((NP, D), jnp.float32),
            pltpu.SemaphoreType.DMA,
        ],
    )
    def edge_sum(src_hbm, dst_hbm, y_hbm, zeros_hbm, out_hbm,
                 src_v, dst_v, rows_v, acc, sem):
        c = lax.axis_index("c")
        s = lax.axis_index("s")
        w = c * 16 + s
        z0 = s * RPT
        pltpu.sync_copy(zeros_hbm.at[pl.ds(z0, RPT)], acc.at[pl.ds(z0, RPT)])
        pltpu.sync_copy(src_hbm.at[w], src_v)
        pltpu.sync_copy(dst_hbm.at[w], dst_v)
        plsc.subcore_barrier()

        def body(i, carry):
            pltpu.async_copy(y_hbm.at[src_v.at[i]], rows_v, sem).wait()
            pltpu.sync_copy(rows_v, acc.at[dst_v.at[i]], add=True)
            return carry

        lax.fori_loop(0, NCH, body, 0)
        plsc.subcore_barrier()
        pltpu.sync_copy(acc.at[pl.ds(z0, RPT)], out_hbm.at[c, pl.ds(z0, RPT)])

    return edge_sum


@functools.partial(
    pl.kernel,
    out_type=jax.ShapeDtypeStruct((2, NP, DW), jnp.float32),
    mesh=_MESH,
    scratch_types=[
        pltpu.VMEM((NCH, CH), jnp.int32),
        pltpu.VMEM((CH, DW), jnp.float32),
        pltpu.VMEM_SHARED((NP, DW), jnp.float32),
    ],
)
def _degree(dst_hbm, zeros_hbm, ones_hbm, out_hbm, dst_v, ones_v, acc):
    c = lax.axis_index("c")
    s = lax.axis_index("s")
    w = c * 16 + s
    z0 = s * RPT
    pltpu.sync_copy(zeros_hbm.at[pl.ds(z0, RPT)], acc.at[pl.ds(z0, RPT)])
    pltpu.sync_copy(dst_hbm.at[w], dst_v)
    pltpu.sync_copy(ones_hbm, ones_v)
    plsc.subcore_barrier()

    def body(i, carry):
        pltpu.sync_copy(ones_v, acc.at[dst_v.at[i]], add=True)
        return carry

    lax.fori_loop(0, NCH, body, 0)
    plsc.subcore_barrier()
    pltpu.sync_copy(acc.at[pl.ds(z0, RPT)], out_hbm.at[c, pl.ds(z0, RPT)])


_edge_sum_128 = _make_edge_sum(128)
_edge_sum_64 = _make_edge_sum(64)


_BLK = 1000
_GRID = (NN // _BLK,)


def _row_spec(d):
    return pl.BlockSpec((_BLK, d), lambda i: (i, 0))


def _full_spec(r, c):
    return pl.BlockSpec((r, c), lambda i: (0, 0))


def _prescale_body(x_ref, w1_ref, c0_ref, c1_ref, y_ref, dinv_ref):
    deg = c0_ref[...] + c1_ref[...] + 1.0
    dinv = lax.rsqrt(deg)
    y_ref[...] = jnp.dot(x_ref[...], w1_ref[...],
                         preferred_element_type=jnp.float32) * dinv
    dinv_ref[...] = dinv


def _tc_prescale(x, w1, cnt0, cnt1):
    return pl.pallas_call(
        _prescale_body,
        grid=_GRID,
        in_specs=[_row_spec(128), _full_spec(128, 128), _row_spec(1), _row_spec(1)],
        out_specs=[_row_spec(128), _row_spec(1)],
        out_shape=[jax.ShapeDtypeStruct((NN, 128), jnp.float32),
                   jax.ShapeDtypeStruct((NN, 1), jnp.float32)],
    )(x, w1, cnt0, cnt1)


def _mid_body(a0_ref, a1_ref, y1_ref, dinv_ref, b1_ref, w2_ref, y2_ref):
    dinv = dinv_ref[...]
    h1 = dinv * (a0_ref[...] + a1_ref[...] + y1_ref[...]) + b1_ref[...]
    h1 = jnp.maximum(h1, 0.0)
    y2_ref[...] = jnp.dot(h1, w2_ref[...],
                          preferred_element_type=jnp.float32) * dinv


def _tc_mid(a0, a1, y1, dinv, b1, w2):
    return pl.pallas_call(
        _mid_body,
        grid=_GRID,
        in_specs=[_row_spec(128), _row_spec(128), _row_spec(128), _row_spec(1),
                  _full_spec(1, 128), _full_spec(128, 64)],
        out_specs=_row_spec(64),
        out_shape=jax.ShapeDtypeStruct((NN, 64), jnp.float32),
    )(a0, a1, y1, dinv, b1, w2)


def _final_body(a0_ref, a1_ref, y2_ref, dinv_ref, b2_ref, wc_ref, bc_ref,
                h2_ref, out_ref):
    h2 = dinv_ref[...] * (a0_ref[...] + a1_ref[...] + y2_ref[...]) + b2_ref[...]
    h2_ref[...] = h2
    out_ref[...] = jnp.dot(h2, wc_ref[...],
                           preferred_element_type=jnp.float32) + bc_ref[...]


def _tc_final(a0, a1, y2, dinv, b2, wc, bc):
    return pl.pallas_call(
        _final_body,
        grid=_GRID,
        in_specs=[_row_spec(64), _row_spec(64), _row_spec(64), _row_spec(1),
                  _full_spec(1, 64), _full_spec(64, 64), _full_spec(1, 64)],
        out_specs=[_row_spec(64), _row_spec(64)],
        out_shape=[jax.ShapeDtypeStruct((NN, 64), jnp.float32),
                   jax.ShapeDtypeStruct((NN, 64), jnp.float32)],
    )(a0, a1, y2, dinv, b2, wc, bc)


def kernel(x, edge_index, W1, b1, W2, b2, Wc, bc):
    src_r = edge_index[0].reshape(NTILES, NCH, CH)
    dst_r = edge_index[1].reshape(NTILES, NCH, CH)

    zeros8 = jnp.zeros((NP, DW), jnp.float32)
    ones8 = jnp.ones((CH, DW), jnp.float32)
    zeros128 = jnp.zeros((NP, 128), jnp.float32)
    zeros64 = jnp.zeros((NP, 64), jnp.float32)

    deg_o = _degree(dst_r, zeros8, ones8)
    cnt0 = deg_o[0, :NN, 0:1]
    cnt1 = deg_o[1, :NN, 0:1]

    y1, dinv = _tc_prescale(x, W1, cnt0, cnt1)

    acc1 = _edge_sum_128(src_r, dst_r, y1, zeros128)
    y2 = _tc_mid(acc1[0, :NN], acc1[1, :NN], y1, dinv, b1.reshape(1, 128), W2)

    acc2 = _edge_sum_64(src_r, dst_r, y2, zeros64)
    out, h2 = _tc_final(acc2[0, :NN], acc2[1, :NN], y2, dinv,
                        b2.reshape(1, 64), Wc, bc.reshape(1, 64))
    return (out, h2)


# speedup vs baseline: 28.2596x; 1.4303x over previous
import functools

import jax
import jax.numpy as jnp
from jax import lax
from jax.experimental import pallas as pl
from jax.experimental.pallas import tpu as pltpu
from jax.experimental.pallas import tpu_sc as plsc

NN = 10000
NP = 10240
EE = 320000
NTILES = 32
EPT = EE // NTILES
CH = 80
NCH = EPT // CH
RPT = NP // 16
DW = 8

_MESH = plsc.VectorSubcoreMesh(core_axis_name="c", subcore_axis_name="s")


def _make_edge_sum(D):
    @functools.partial(
        pl.kernel,
        out_type=jax.ShapeDtypeStruct((2, NP, D), jnp.float32),
        mesh=_MESH,
        compiler_params=pltpu.CompilerParams(use_tc_tiling_on_sc=False),
        scratch_types=[
            pltpu.VMEM((NCH, CH), jnp.int32),
            pltpu.VMEM((NCH, CH), jnp.int32),
            pltpu.VMEM((CH, D), jnp.float32),
            pltpu.VMEM((CH, D), jnp.float32),
            pltpu.VMEM_SHARED((NP, D), jnp.float32),
            pltpu.SemaphoreType.DMA,
            pltpu.SemaphoreType.DMA,
        ],
    )
    def edge_sum(src_hbm, dst_hbm, y_hbm, zeros_hbm, out_hbm,
                 src_v, dst_v, rows_a, rows_b, acc, sem_a, sem_b):
        c = lax.axis_index("c")
        s = lax.axis_index("s")
        w = c * 16 + s
        z0 = s * RPT
        pltpu.sync_copy(zeros_hbm.at[pl.ds(z0, RPT)], acc.at[pl.ds(z0, RPT)])
        pltpu.sync_copy(src_hbm.at[w], src_v)
        pltpu.sync_copy(dst_hbm.at[w], dst_v)
        plsc.subcore_barrier()

        pltpu.async_copy(y_hbm.at[src_v.at[0]], rows_a, sem_a)

        def body(j, carry):
            c0 = 2 * j
            pltpu.async_copy(y_hbm.at[src_v.at[c0 + 1]], rows_b, sem_b)
            pltpu.make_async_copy(y_hbm.at[src_v.at[c0]], rows_a, sem_a).wait()
            pltpu.sync_copy(rows_a, acc.at[dst_v.at[c0]], add=True)
            pltpu.async_copy(y_hbm.at[src_v.at[c0 + 2]], rows_a, sem_a)
            pltpu.make_async_copy(y_hbm.at[src_v.at[c0 + 1]], rows_b, sem_b).wait()
            pltpu.sync_copy(rows_b, acc.at[dst_v.at[c0 + 1]], add=True)
            return carry

        lax.fori_loop(0, (NCH - 1) // 2, body, 0)
        pltpu.make_async_copy(y_hbm.at[src_v.at[NCH - 1]], rows_a, sem_a).wait()
        pltpu.sync_copy(rows_a, acc.at[dst_v.at[NCH - 1]], add=True)
        plsc.subcore_barrier()
        pltpu.sync_copy(acc.at[pl.ds(z0, RPT)], out_hbm.at[c, pl.ds(z0, RPT)])

    return edge_sum


@functools.partial(
    pl.kernel,
    out_type=jax.ShapeDtypeStruct((2, NP, DW), jnp.float32),
    mesh=_MESH,
    scratch_types=[
        pltpu.VMEM((NCH, CH), jnp.int32),
        pltpu.VMEM((CH, DW), jnp.float32),
        pltpu.VMEM_SHARED((NP, DW), jnp.float32),
    ],
)
def _degree(dst_hbm, zeros_hbm, ones_hbm, out_hbm, dst_v, ones_v, acc):
    c = lax.axis_index("c")
    s = lax.axis_index("s")
    w = c * 16 + s
    z0 = s * RPT
    pltpu.sync_copy(zeros_hbm.at[pl.ds(z0, RPT)], acc.at[pl.ds(z0, RPT)])
    pltpu.sync_copy(dst_hbm.at[w], dst_v)
    pltpu.sync_copy(ones_hbm, ones_v)
    plsc.subcore_barrier()

    def body(i, carry):
        pltpu.sync_copy(ones_v, acc.at[dst_v.at[i]], add=True)
        return carry

    lax.fori_loop(0, NCH, body, 0)
    plsc.subcore_barrier()
    pltpu.sync_copy(acc.at[pl.ds(z0, RPT)], out_hbm.at[c, pl.ds(z0, RPT)])


_edge_sum_128 = _make_edge_sum(128)
_edge_sum_64 = _make_edge_sum(64)


_BLK = 1000
_GRID = (NN // _BLK,)


def _row_spec(d):
    return pl.BlockSpec((_BLK, d), lambda i: (i, 0))


def _full_spec(r, c):
    return pl.BlockSpec((r, c), lambda i: (0, 0))


def _prescale_body(x_ref, w1_ref, c0_ref, c1_ref, y_ref, dinv_ref):
    deg = c0_ref[...] + c1_ref[...] + 1.0
    dinv = lax.rsqrt(deg)
    y_ref[...] = jnp.dot(x_ref[...], w1_ref[...],
                         preferred_element_type=jnp.float32) * dinv
    dinv_ref[...] = dinv


def _tc_prescale(x, w1, cnt0, cnt1):
    return pl.pallas_call(
        _prescale_body,
        grid=_GRID,
        in_specs=[_row_spec(128), _full_spec(128, 128), _row_spec(1), _row_spec(1)],
        out_specs=[_row_spec(128), _row_spec(1)],
        out_shape=[jax.ShapeDtypeStruct((NN, 128), jnp.float32),
                   jax.ShapeDtypeStruct((NN, 1), jnp.float32)],
    )(x, w1, cnt0, cnt1)


def _mid_body(a0_ref, a1_ref, y1_ref, dinv_ref, b1_ref, w2_ref, y2_ref):
    dinv = dinv_ref[...]
    h1 = dinv * (a0_ref[...] + a1_ref[...] + y1_ref[...]) + b1_ref[...]
    h1 = jnp.maximum(h1, 0.0)
    y2_ref[...] = jnp.dot(h1, w2_ref[...],
                          preferred_element_type=jnp.float32) * dinv


def _tc_mid(a0, a1, y1, dinv, b1, w2):
    return pl.pallas_call(
        _mid_body,
        grid=_GRID,
        in_specs=[_row_spec(128), _row_spec(128), _row_spec(128), _row_spec(1),
                  _full_spec(1, 128), _full_spec(128, 64)],
        out_specs=_row_spec(64),
        out_shape=jax.ShapeDtypeStruct((NN, 64), jnp.float32),
    )(a0, a1, y1, dinv, b1, w2)


def _final_body(a0_ref, a1_ref, y2_ref, dinv_ref, b2_ref, wc_ref, bc_ref,
                h2_ref, out_ref):
    h2 = dinv_ref[...] * (a0_ref[...] + a1_ref[...] + y2_ref[...]) + b2_ref[...]
    h2_ref[...] = h2
    out_ref[...] = jnp.dot(h2, wc_ref[...],
                           preferred_element_type=jnp.float32) + bc_ref[...]


def _tc_final(a0, a1, y2, dinv, b2, wc, bc):
    return pl.pallas_call(
        _final_body,
        grid=_GRID,
        in_specs=[_row_spec(64), _row_spec(64), _row_spec(64), _row_spec(1),
                  _full_spec(1, 64), _full_spec(64, 64), _full_spec(1, 64)],
        out_specs=[_row_spec(64), _row_spec(64)],
        out_shape=[jax.ShapeDtypeStruct((NN, 64), jnp.float32),
                   jax.ShapeDtypeStruct((NN, 64), jnp.float32)],
    )(a0, a1, y2, dinv, b2, wc, bc)


def kernel(x, edge_index, W1, b1, W2, b2, Wc, bc):
    src_r = edge_index[0].reshape(NTILES, NCH, CH)
    dst_r = edge_index[1].reshape(NTILES, NCH, CH)

    zeros8 = jnp.zeros((NP, DW), jnp.float32)
    ones8 = jnp.ones((CH, DW), jnp.float32)
    zeros128 = jnp.zeros((NP, 128), jnp.float32)
    zeros64 = jnp.zeros((NP, 64), jnp.float32)

    deg_o = _degree(dst_r, zeros8, ones8)
    cnt0 = deg_o[0, :NN, 0:1]
    cnt1 = deg_o[1, :NN, 0:1]

    y1, dinv = _tc_prescale(x, W1, cnt0, cnt1)

    acc1 = _edge_sum_128(src_r, dst_r, y1, zeros128)
    y2 = _tc_mid(acc1[0, :NN], acc1[1, :NN], y1, dinv, b1.reshape(1, 128), W2)

    acc2 = _edge_sum_64(src_r, dst_r, y2, zeros64)
    out, h2 = _tc_final(acc2[0, :NN], acc2[1, :NN], y2, dinv,
                        b2.reshape(1, 64), Wc, bc.reshape(1, 64))
    return (out, h2)
